# baseline (speedup 1.0000x reference)
import functools

import jax
import jax.numpy as jnp
from jax import lax
from jax.experimental import pallas as pl
from jax.experimental.pallas import tpu as pltpu
from jax.experimental.pallas import tpu_sc as plsc

N_DET = 5000
M_REF = 1000
C_IN = 256
SPATIAL = 49
FC_OUT = 1024

BN = 200
BM = 200

_GROUP = 8
_NGROUPS = N_DET // _GROUP
_NWORKERS = 32
_KMAX = -(-_NGROUPS // _NWORKERS)
_W1P = 1008
_W2P = 2016


def _pool_fc(xb, w1, b1, w2, b2):
    pooled = jnp.sum(xb, axis=0) / 49.0
    h = jnp.maximum(jnp.dot(pooled, w1, preferred_element_type=jnp.float32) + b1, 0.0)
    return jnp.dot(h, w2, preferred_element_type=jnp.float32) + b2


def _ref_kernel(rx_ref, w1_ref, b1_ref, w2_ref, b2_ref, rf_ref):
    rf_ref[...] = _pool_fc(rx_ref[...], w1_ref[...], b1_ref[...],
                           w2_ref[...], b2_ref[...])


def _main_kernel(x_ref, bb_ref, w1_ref, b1_ref, w2_ref, b2_ref,
                 rf_ref, refg_ref, prod_ref, ious_ref, dxy_ref):
    xf = _pool_fc(x_ref[...], w1_ref[...], b1_ref[...], w2_ref[...], b2_ref[...])
    prod_ref[...] = lax.dot_general(
        xf, rf_ref[...], (((1,), (1,)), ((), ())),
        preferred_element_type=jnp.float32)

    bb = bb_ref[...]
    x1 = bb[:, 0:1]
    y1 = bb[:, 1:2]
    x2 = bb[:, 2:3]
    y2 = bb[:, 3:4]
    cx = (x1 + x2) / 2.0
    cy = (y1 + y2) / 2.0
    rg = refg_ref[...]
    rx1 = rg[0:1, :]
    ry1 = rg[1:2, :]
    rx2 = rg[2:3, :]
    ry2 = rg[3:4, :]
    rcx = rg[4:5, :]
    rcy = rg[5:6, :]
    areab = rg[6:7, :]
    dx = rcx - cx
    dy = rcy - cy
    sx1 = x1 + dx
    sy1 = y1 + dy
    sx2 = x2 + dx
    sy2 = y2 + dy
    wx = jnp.maximum(jnp.minimum(sx2, rx2) - jnp.maximum(sx1, rx1), 0.0)
    wy = jnp.maximum(jnp.minimum(sy2, ry2) - jnp.maximum(sy1, ry1), 0.0)
    ov = wx * wy
    areaa = (sx2 - sx1) * (sy2 - sy1)
    union = areaa + areab - ov
    ious_ref[...] = ov / jnp.maximum(union, 1e-6)

    col = jax.lax.broadcasted_iota(jnp.int32, (BN, M_REF + 1), 1)
    keep = col >= 1
    dxy_ref[0] = jnp.where(keep, dx, 0.0)
    dxy_ref[1] = jnp.where(keep, dy, 0.0)


_W1 = M_REF + 1
_W2 = 2 * M_REF + 2
_DSP_ST = _GROUP * _W2
_DX_ST = _GROUP * _W1


def _sc_distances(cxy_hbm, rcil_hbm, tails_hbm, dsp_hbm,
                  cxy_v, rcil_v, tails_v,
                  dspa_st, dspb_st, sema, semb):
    w = lax.axis_index("s") * 2 + lax.axis_index("c")
    gw_start = w * _KMAX
    pltpu.sync_copy(rcil_hbm, rcil_v)
    pltpu.sync_copy(tails_hbm, tails_v)
    pltpu.sync_copy(
        cxy_hbm.at[pl.ds(gw_start * _GROUP * 32, _KMAX * _GROUP * 32)], cxy_v)
    l16 = lax.iota(jnp.int32, 16)
    par = (l16 & 1) == 0
    m2 = l16 >= 2

    def compute_group(g, dsp_st, sem):
        lbase = (g - gw_start) * _GROUP

        @plsc.parallel_loop(0, _GROUP)
        def row_body(r):
            cxs = cxy_v[pl.ds((lbase + r) * 32, 16)]
            cys = cxy_v[pl.ds((lbase + r) * 32 + 16, 16)]
            csel = jnp.where(par, cxs, cys)
            od = r * _W2

            @plsc.parallel_loop(1, _W2 // 16, unroll=8)
            def dsp_chunk(c):
                dsp_st[pl.ds(od + c * 16, 16)] = rcil_v[pl.ds(c * 16, 16)] - csel
            dsp_st[pl.ds(od, 16)] = jnp.where(
                m2, rcil_v[pl.ds(0, 16)] - csel, 0.0)
            dsp_st[pl.ds(od + (_W2 - 16), 16)] = tails_v[pl.ds(0, 16)] - csel

        pltpu.async_copy(dsp_st, dsp_hbm.at[pl.ds(g * _DSP_ST, _DSP_ST)], sem)

    def drain(dsp_st, sem):
        pltpu.make_async_copy(dsp_st, dsp_hbm.at[pl.ds(0, _DSP_ST)], sem).wait()

    def pair_body(k, carry):
        ga = gw_start + 2 * k
        gb = ga + 1

        @pl.when(ga < _NGROUPS)
        def _():
            @pl.when(k > 0)
            def _():
                drain(dspa_st, sema)
            compute_group(ga, dspa_st, sema)

        @pl.when(gb < _NGROUPS)
        def _():
            @pl.when(k > 0)
            def _():
                drain(dspb_st, semb)
            compute_group(gb, dspb_st, semb)

        return carry

    lax.fori_loop(0, (_KMAX + 1) // 2, pair_body, 0)
    drain(dspa_st, sema)
    drain(dspb_st, semb)


def kernel(bboxes, ref_bboxes, x, ref_x, x_n, ref_x_n, W1, b1, W2, b2):
    del x_n, ref_x_n
    xt = jnp.transpose(x, (2, 3, 0, 1)).reshape(SPATIAL, N_DET, C_IN)
    rxt = jnp.transpose(ref_x, (2, 3, 0, 1)).reshape(SPATIAL, M_REF, C_IN)
    b1r = b1.reshape(1, FC_OUT)
    b2r = b2.reshape(1, FC_OUT)

    rf = pl.pallas_call(
        _ref_kernel,
        grid=(M_REF // BM,),
        in_specs=[
            pl.BlockSpec((SPATIAL, BM, C_IN), lambda i: (0, i, 0)),
            pl.BlockSpec((C_IN, FC_OUT), lambda i: (0, 0)),
            pl.BlockSpec((1, FC_OUT), lambda i: (0, 0)),
            pl.BlockSpec((FC_OUT, FC_OUT), lambda i: (0, 0)),
            pl.BlockSpec((1, FC_OUT), lambda i: (0, 0)),
        ],
        out_specs=pl.BlockSpec((BM, FC_OUT), lambda i: (i, 0)),
        out_shape=jax.ShapeDtypeStruct((M_REF, FC_OUT), jnp.float32),
        compiler_params=pltpu.CompilerParams(dimension_semantics=("arbitrary",), skip_device_barrier=True),
    )(rxt, W1, b1r, W2, b2r)

    rcx = (ref_bboxes[:, 0] + ref_bboxes[:, 2]) / 2.0
    rcy = (ref_bboxes[:, 1] + ref_bboxes[:, 3]) / 2.0
    areab = (ref_bboxes[:, 2] - ref_bboxes[:, 0]) * (ref_bboxes[:, 3] - ref_bboxes[:, 1])
    refg = jnp.pad(
        jnp.stack([ref_bboxes[:, 0], ref_bboxes[:, 1], ref_bboxes[:, 2],
                   ref_bboxes[:, 3], rcx, rcy, areab,
                   jnp.zeros((M_REF,), jnp.float32)], axis=0),
        ((0, 0), (1, 0)))

    rcil = jnp.pad(jnp.stack([rcx, rcy], axis=1).reshape(-1),
                   (2, _W2P - 2 * M_REF - 2))
    tails = rcil[_W2 - 16:_W2]
    cx = (bboxes[:, 0] + bboxes[:, 2]) / 2.0
    cy = (bboxes[:, 1] + bboxes[:, 3]) / 2.0
    cxy_rep = jnp.pad(
        jnp.broadcast_to(
            jnp.stack([cx, cy], axis=1)[:, :, None], (N_DET, 2, 16)
        ).reshape(N_DET * 32),
        (0, (_NWORKERS * _KMAX * _GROUP - N_DET) * 32))

    sck = functools.partial(
        pl.kernel,
        out_type=jax.ShapeDtypeStruct((N_DET * _W2,), jnp.float32),
        mesh=plsc.VectorSubcoreMesh(core_axis_name="c", subcore_axis_name="s"),
        compiler_params=pltpu.CompilerParams(needs_layout_passes=False),
        scratch_types=[
            pltpu.VMEM((_KMAX * _GROUP * 32,), jnp.float32),
            pltpu.VMEM((_W2P,), jnp.float32),
            pltpu.VMEM((16,), jnp.float32),
            pltpu.VMEM((_DSP_ST,), jnp.float32),
            pltpu.VMEM((_DSP_ST,), jnp.float32),
            pltpu.SemaphoreType.DMA,
            pltpu.SemaphoreType.DMA,
        ],
    )(_sc_distances)
    dsp1 = sck(cxy_rep, rcil, tails)
    dsp = dsp1.reshape(1, N_DET, _W2)

    prod, ious2, dxy = pl.pallas_call(
        _main_kernel,
        grid=(N_DET // BN,),
        in_specs=[
            pl.BlockSpec((SPATIAL, BN, C_IN), lambda i: (0, i, 0)),
            pl.BlockSpec((BN, 4), lambda i: (i, 0)),
            pl.BlockSpec((C_IN, FC_OUT), lambda i: (0, 0)),
            pl.BlockSpec((1, FC_OUT), lambda i: (0, 0)),
            pl.BlockSpec((FC_OUT, FC_OUT), lambda i: (0, 0)),
            pl.BlockSpec((1, FC_OUT), lambda i: (0, 0)),
            pl.BlockSpec((M_REF, FC_OUT), lambda i: (0, 0)),
            pl.BlockSpec((8, M_REF + 1), lambda i: (0, 0)),
        ],
        out_specs=[
            pl.BlockSpec((BN, M_REF), lambda i: (i, 0)),
            pl.BlockSpec((BN, M_REF + 1), lambda i: (i, 0)),
            pl.BlockSpec((2, BN, M_REF + 1), lambda i: (0, i, 0)),
        ],
        out_shape=[
            jax.ShapeDtypeStruct((N_DET, M_REF), jnp.float32),
            jax.ShapeDtypeStruct((N_DET, M_REF + 1), jnp.float32),
            jax.ShapeDtypeStruct((2, N_DET, M_REF + 1), jnp.float32),
        ],
        compiler_params=pltpu.CompilerParams(dimension_semantics=("arbitrary",), skip_device_barrier=True),
    )(xt, bboxes, W1, b1r, W2, b2r, rf, refg)

    return prod, ious2, dxy, dsp


# speedup vs baseline: 1.4284x; 1.4284x over previous
import jax
import jax.numpy as jnp
from jax.experimental import pallas as pl
from jax.experimental.pallas import tpu as pltpu

N_DET = 5000
M_REF = 1000
C_IN = 256
SPATIAL = 49
FC_OUT = 1024

BN = 200
BM = 200


def _pool_fc(xb, w1, b1, w2, b2):
    pooled = jnp.sum(xb, axis=0) / 49.0
    h = jnp.maximum(jnp.dot(pooled, w1, preferred_element_type=jnp.float32) + b1, 0.0)
    return jnp.dot(h, w2, preferred_element_type=jnp.float32) + b2


def _ref_kernel(rx_ref, w1_ref, b1_ref, w2_ref, b2_ref, rf_ref):
    rf_ref[...] = _pool_fc(rx_ref[...], w1_ref[...], b1_ref[...],
                           w2_ref[...], b2_ref[...])


def _main_kernel(x_ref, bb_ref, w1_ref, b1_ref, w2_ref, b2_ref,
                 rft_ref, refg_ref, rcil_ref,
                 prod_ref, ious_ref, dxy_ref, dsp_ref):
    xf = _pool_fc(x_ref[...], w1_ref[...], b1_ref[...], w2_ref[...], b2_ref[...])
    prod_ref[...] = jax.lax.dot_general(
        xf, rft_ref[...], (((1,), (1,)), ((), ())),
        preferred_element_type=jnp.float32)

    bb = bb_ref[...]
    x1 = bb[:, 0:1]
    y1 = bb[:, 1:2]
    x2 = bb[:, 2:3]
    y2 = bb[:, 3:4]
    cx = (x1 + x2) / 2.0
    cy = (y1 + y2) / 2.0
    rg = refg_ref[...]
    rx1 = rg[0:1, :]
    ry1 = rg[1:2, :]
    rx2 = rg[2:3, :]
    ry2 = rg[3:4, :]
    rcx = rg[4:5, :]
    rcy = rg[5:6, :]
    areab = rg[6:7, :]
    dx = rcx - cx
    dy = rcy - cy
    sx1 = x1 + dx
    sy1 = y1 + dy
    sx2 = x2 + dx
    sy2 = y2 + dy
    wx = jnp.maximum(jnp.minimum(sx2, rx2) - jnp.maximum(sx1, rx1), 0.0)
    wy = jnp.maximum(jnp.minimum(sy2, ry2) - jnp.maximum(sy1, ry1), 0.0)
    ov = wx * wy
    areaa = (sx2 - sx1) * (sy2 - sy1)
    union = areaa + areab - ov
    ious_ref[...] = ov / jnp.maximum(union, 1e-6)

    col = jax.lax.broadcasted_iota(jnp.int32, (BN, M_REF + 1), 1)
    keep = col >= 1
    dxy_ref[0] = jnp.where(keep, dx, 0.0)
    dxy_ref[1] = jnp.where(keep, dy, 0.0)

    col2 = jax.lax.broadcasted_iota(jnp.int32, (BN, 2 * M_REF + 2), 1)
    c_il = jnp.where((col2 & 1) == 0, cx, cy)
    dsp_ref[0] = jnp.where(col2 >= 2, rcil_ref[...] - c_il, 0.0)


def kernel(bboxes, ref_bboxes, x, ref_x, x_n, ref_x_n, W1, b1, W2, b2):
    del x_n, ref_x_n
    xt = jnp.transpose(x, (2, 3, 0, 1)).reshape(SPATIAL, N_DET, C_IN)
    rxt = jnp.transpose(ref_x, (2, 3, 0, 1)).reshape(SPATIAL, M_REF, C_IN)
    b1r = b1.reshape(1, FC_OUT)
    b2r = b2.reshape(1, FC_OUT)

    rf = pl.pallas_call(
        _ref_kernel,
        grid=(M_REF // BM,),
        in_specs=[
            pl.BlockSpec((SPATIAL, BM, C_IN), lambda i: (0, i, 0)),
            pl.BlockSpec((C_IN, FC_OUT), lambda i: (0, 0)),
            pl.BlockSpec((1, FC_OUT), lambda i: (0, 0)),
            pl.BlockSpec((FC_OUT, FC_OUT), lambda i: (0, 0)),
            pl.BlockSpec((1, FC_OUT), lambda i: (0, 0)),
        ],
        out_specs=pl.BlockSpec((BM, FC_OUT), lambda i: (i, 0)),
        out_shape=jax.ShapeDtypeStruct((M_REF, FC_OUT), jnp.float32),
        compiler_params=pltpu.CompilerParams(dimension_semantics=("arbitrary",)),
    )(rxt, W1, b1r, W2, b2r)

    rcx = (ref_bboxes[:, 0] + ref_bboxes[:, 2]) / 2.0
    rcy = (ref_bboxes[:, 1] + ref_bboxes[:, 3]) / 2.0
    areab = (ref_bboxes[:, 2] - ref_bboxes[:, 0]) * (ref_bboxes[:, 3] - ref_bboxes[:, 1])
    refg = jnp.pad(
        jnp.stack([ref_bboxes[:, 0], ref_bboxes[:, 1], ref_bboxes[:, 2],
                   ref_bboxes[:, 3], rcx, rcy, areab,
                   jnp.zeros((M_REF,), jnp.float32)], axis=0),
        ((0, 0), (1, 0)))
    rcil = jnp.concatenate(
        [jnp.zeros((2,), jnp.float32), jnp.stack([rcx, rcy], axis=1).reshape(-1)]
    ).reshape(1, 2 * M_REF + 2)

    prod, ious2, dxy, dsp = pl.pallas_call(
        _main_kernel,
        grid=(N_DET // BN,),
        in_specs=[
            pl.BlockSpec((SPATIAL, BN, C_IN), lambda i: (0, i, 0)),
            pl.BlockSpec((BN, 4), lambda i: (i, 0)),
            pl.BlockSpec((C_IN, FC_OUT), lambda i: (0, 0)),
            pl.BlockSpec((1, FC_OUT), lambda i: (0, 0)),
            pl.BlockSpec((FC_OUT, FC_OUT), lambda i: (0, 0)),
            pl.BlockSpec((1, FC_OUT), lambda i: (0, 0)),
            pl.BlockSpec((M_REF, FC_OUT), lambda i: (0, 0)),
            pl.BlockSpec((8, M_REF + 1), lambda i: (0, 0)),
            pl.BlockSpec((1, 2 * M_REF + 2), lambda i: (0, 0)),
        ],
        out_specs=[
            pl.BlockSpec((BN, M_REF), lambda i: (i, 0)),
            pl.BlockSpec((BN, M_REF + 1), lambda i: (i, 0)),
            pl.BlockSpec((2, BN, M_REF + 1), lambda i: (0, i, 0)),
            pl.BlockSpec((1, BN, 2 * M_REF + 2), lambda i: (0, i, 0)),
        ],
        out_shape=[
            jax.ShapeDtypeStruct((N_DET, M_REF), jnp.float32),
            jax.ShapeDtypeStruct((N_DET, M_REF + 1), jnp.float32),
            jax.ShapeDtypeStruct((2, N_DET, M_REF + 1), jnp.float32),
            jax.ShapeDtypeStruct((1, N_DET, 2 * M_REF + 2), jnp.float32),
        ],
        compiler_params=pltpu.CompilerParams(dimension_semantics=("arbitrary",)),
    )(xt, bboxes, W1, b1r, W2, b2r, rf, refg, rcil)

    return prod, ious2, dxy, dsp
